# Initial kernel scaffold; baseline (speedup 1.0000x reference)
import jax
import jax.numpy as jnp
from jax.experimental import pallas as pl


def kernel(node_features, edge_src, edge_features, edge_len, seq_times, t_w0, t_b0, t_w, t_b, e_wih, e_whh, e_bih, e_bhh, a_wih, a_whh, a_bih, a_bhh, w_attn, w_eout, b_eout, w_node, b_node):
    raise NotImplementedError("write your pallas kernel here")


# speedup vs baseline: 3.3895x; 3.3895x over previous
import functools

import jax
import jax.numpy as jnp
from jax import lax
from jax.experimental import pallas as pl
from jax.experimental.pallas import tpu as pltpu
from jax.experimental.pallas import tpu_sc as plsc

H = 64
T = 8
DEG = 16
EDGE_IN = 16
DT = 16
NODE_IN = 128


def _make_sc_gather(v, d, b):
    info = plsc.get_sparse_core_info()
    nw = info.num_cores * info.num_subcores
    b_per_w = b // nw
    chunk = 1000
    n_chunks = b_per_w // chunk
    mesh = plsc.VectorSubcoreMesh(core_axis_name="c", subcore_axis_name="s")

    @functools.partial(
        pl.kernel,
        mesh=mesh,
        out_type=jax.ShapeDtypeStruct((b, d), jnp.float32),
        scratch_types=[
            pltpu.VMEM((chunk,), jnp.int32),
            pltpu.VMEM((chunk, d), jnp.float32),
            pltpu.SemaphoreType.DMA,
        ],
    )
    def k(table_hbm, idx_hbm, out_hbm, idx_v, rows_v, sem):
        wid = lax.axis_index("s") * info.num_cores + lax.axis_index("c")
        base = wid * b_per_w
        for j in range(n_chunks):
            off = base + j * chunk
            pltpu.sync_copy(idx_hbm.at[pl.ds(off, chunk)], idx_v)
            pltpu.async_copy(table_hbm.at[idx_v], rows_v, sem).wait()
            pltpu.sync_copy(rows_v, out_hbm.at[pl.ds(off, chunk)])

    return k


def _main_body(ef_ref, tt_ref, mk_ref, ga_ref, nf_ref,
               wih_ref, whh_ref, b_ref, wexp_ref, bexp_ref,
               wat_ref, weon_ref, weo_ref, beo_ref, wnon_ref, wno_ref, bno_ref, out_ref):
    ef = ef_ref[...]
    tt = tt_ref[...]
    mk = mk_ref[...]
    be = ef.shape[0]
    bn = be // DEG

    tv_lin = jnp.dot(tt, wexp_ref[...], preferred_element_type=jnp.float32) + bexp_ref[...]
    lane = lax.broadcasted_iota(jnp.int32, (1, T * DT), 1)
    tv = jnp.where((lane % DT) < (DT - 1), jnp.sin(tv_lin), tv_lin)

    wih = wih_ref[...]
    whh = whh_ref[...]
    bias = b_ref[...]
    h = jnp.zeros((be, 2 * H), jnp.float32)
    c = jnp.zeros((be, 2 * H), jnp.float32)
    h_last = jnp.zeros((be, 2 * H), jnp.float32)
    for t in range(T):
        x_t = jnp.concatenate(
            [ef[:, EDGE_IN * t:EDGE_IN * (t + 1)], tv[:, DT * t:DT * (t + 1)]], axis=1)
        g = (jnp.dot(x_t, wih, preferred_element_type=jnp.float32)
             + jnp.dot(h, whh, preferred_element_type=jnp.float32) + bias)
        gi = jax.nn.sigmoid(g[:, 0:128])
        gf = jax.nn.sigmoid(g[:, 128:256])
        gg = jnp.tanh(g[:, 256:384])
        go = jax.nn.sigmoid(g[:, 384:512])
        c = gf * c + gi * gg
        h = go * jnp.tanh(c)
        h_last = jnp.where(mk[:, t:t + 1] > 0, h, h_last)

    e_out = h_last[:, 0:H]
    a_h = h_last[:, H:2 * H]
    a = jnp.sum(a_h * wat_ref[...], axis=1, keepdims=True)
    a = jnp.where(a >= 0, a, 0.01 * a)
    m = jnp.maximum(
        jnp.dot(ga_ref[...], weon_ref[...], preferred_element_type=jnp.float32)
        + jnp.dot(e_out, weo_ref[...], preferred_element_type=jnp.float32)
        + beo_ref[...], 0.0)

    erow = lax.broadcasted_iota(jnp.int32, (be, DEG), 0)
    ecol = lax.broadcasted_iota(jnp.int32, (be, DEG), 1)
    p_oh = (erow % DEG == ecol).astype(jnp.float32)
    gn = lax.broadcasted_iota(jnp.int32, (bn, be), 0)
    ge = lax.broadcasted_iota(jnp.int32, (bn, be), 1)
    g_oh = (ge // DEG == gn).astype(jnp.float32)
    gn2 = lax.broadcasted_iota(jnp.int32, (be, bn), 1)
    ge2 = lax.broadcasted_iota(jnp.int32, (be, bn), 0)
    gt_oh = (ge2 // DEG == gn2).astype(jnp.float32)

    z = jnp.dot(g_oh, a * p_oh, preferred_element_type=jnp.float32)
    z = z - jnp.max(z, axis=1, keepdims=True)
    cmp = (z[:, None, :] >= z[:, :, None]).astype(jnp.float32)
    r = jnp.sum(cmp, axis=2)
    s = jnp.sum(cmp * z[:, None, :], axis=2)
    valid = (1.0 + r * z) > s
    tauc = jnp.where(valid, (s - 1.0) / r, -1e30)
    tau = jnp.max(tauc, axis=1, keepdims=True)
    alpha = jnp.maximum(z - tau, 0.0)

    aedge = jnp.sum(jnp.dot(gt_oh, alpha, preferred_element_type=jnp.float32) * p_oh,
                    axis=1, keepdims=True)
    h_neigh = jnp.dot(g_oh, aedge * m, preferred_element_type=jnp.float32)
    out_ref[...] = jnp.maximum(
        jnp.dot(nf_ref[...], wnon_ref[...], preferred_element_type=jnp.float32)
        + jnp.dot(h_neigh, wno_ref[...], preferred_element_type=jnp.float32)
        + bno_ref[...], 0.0)


def _main_call(ef2, tt, mk, ga, nf, wih, whh, bias, wexp, bexp,
               wat, weon, weo, beo, wnon, wno, bno, bn):
    e = ef2.shape[0]
    n = nf.shape[0]
    be = DEG * bn

    def full(shape):
        return pl.BlockSpec(shape, lambda i: tuple(0 for _ in shape))

    return pl.pallas_call(
        _main_body,
        grid=(e // be,),
        in_specs=[
            pl.BlockSpec((be, T * EDGE_IN), lambda i: (i, 0)),
            pl.BlockSpec((be, T), lambda i: (i, 0)),
            pl.BlockSpec((be, T), lambda i: (i, 0)),
            pl.BlockSpec((be, NODE_IN), lambda i: (i, 0)),
            pl.BlockSpec((bn, NODE_IN), lambda i: (i, 0)),
            full((2 * EDGE_IN, 8 * H)),
            full((2 * H, 8 * H)),
            full((1, 8 * H)),
            full((T, T * DT)),
            full((1, T * DT)),
            full((1, H)),
            full((NODE_IN, H)),
            full((H, H)),
            full((1, H)),
            full((NODE_IN, H)),
            full((H, H)),
            full((1, H)),
        ],
        out_specs=pl.BlockSpec((bn, H), lambda i: (i, 0)),
        out_shape=jax.ShapeDtypeStruct((n, H), jnp.float32),
    )(ef2, tt, mk, ga, nf, wih, whh, bias, wexp, bexp, wat, weon, weo, beo, wnon, wno, bno)


def _interleave_gates(we, wa):
    blocks = []
    for k in range(4):
        blocks.append(we.T[:, H * k:H * (k + 1)])
        blocks.append(wa.T[:, H * k:H * (k + 1)])
    return jnp.concatenate(blocks, axis=1)


def kernel(node_features, edge_src, edge_features, edge_len, seq_times,
           t_w0, t_b0, t_w, t_b,
           e_wih, e_whh, e_bih, e_bhh,
           a_wih, a_whh, a_bih, a_bhh,
           w_attn, w_eout, b_eout, w_node, b_node):
    e = edge_features.shape[0]
    n = node_features.shape[0]

    ef2 = edge_features.reshape(e, T * EDGE_IN)
    mask = (jnp.arange(T, dtype=jnp.int32)[None, :] < edge_len[:, None]).astype(jnp.float32)

    wrow = jnp.concatenate([t_w[0], t_w0[0]])
    wexp = jnp.kron(jnp.eye(T, dtype=jnp.float32), wrow[None, :])
    bexp = jnp.tile(jnp.concatenate([t_b, t_b0]), T)[None, :]

    wih_cat = _interleave_gates(e_wih, a_wih)
    z64 = jnp.zeros((H, H), jnp.float32)
    top, bot = [], []
    for k in range(4):
        top += [e_whh.T[:, H * k:H * (k + 1)], z64]
        bot += [z64, a_whh.T[:, H * k:H * (k + 1)]]
    whh_bd = jnp.concatenate(
        [jnp.concatenate(top, axis=1), jnp.concatenate(bot, axis=1)], axis=0)
    be_, ba_ = e_bih + e_bhh, a_bih + a_bhh
    bias = jnp.concatenate(
        [jnp.concatenate([be_[H * k:H * (k + 1)], ba_[H * k:H * (k + 1)]]) for k in range(4)]
    )[None, :]

    weon = w_eout[:, :NODE_IN].T
    weo = w_eout[:, NODE_IN:].T
    wnon = w_node[:, :NODE_IN].T
    wno = w_node[:, NODE_IN:].T

    gathered = _sc_gather(node_features, edge_src, e)

    return _main_call(ef2, seq_times, mask, gathered, node_features,
                      wih_cat, whh_bd, bias, wexp, bexp,
                      w_attn, weon, weo, b_eout[None, :], wnon, wno, b_node[None, :], bn=80)


def _sc_gather(table, idx, e):
    return _make_sc_gather(table.shape[0], table.shape[1], e)(table, idx)


# speedup vs baseline: 5.1790x; 1.5280x over previous
import functools

import jax
import jax.numpy as jnp
from jax import lax
from jax.experimental import pallas as pl
from jax.experimental.pallas import tpu as pltpu
from jax.experimental.pallas import tpu_sc as plsc

H = 64
T = 8
DEG = 16
EDGE_IN = 16
DT = 16
NODE_IN = 128


def _make_sc_gather(v, d, b, goff):
    info = plsc.get_sparse_core_info()
    nw = info.num_cores * info.num_subcores
    b_per_w = b // nw
    chunk = 1000
    n_chunks = b_per_w // chunk
    mesh = plsc.VectorSubcoreMesh(core_axis_name="c", subcore_axis_name="s")

    @functools.partial(
        pl.kernel,
        mesh=mesh,
        out_type=jax.ShapeDtypeStruct((b, d), jnp.float32),
        scratch_types=[
            pltpu.VMEM((chunk,), jnp.int32),
            pltpu.VMEM((chunk, d), jnp.float32),
            pltpu.SemaphoreType.DMA,
        ],
    )
    def k(table_hbm, idx_hbm, out_hbm, idx_v, rows_v, sem):
        wid = lax.axis_index("s") * info.num_cores + lax.axis_index("c")
        base = wid * b_per_w
        for j in range(n_chunks):
            off = base + j * chunk
            pltpu.sync_copy(idx_hbm.at[pl.ds(goff + off, chunk)], idx_v)
            pltpu.async_copy(table_hbm.at[idx_v], rows_v, sem).wait()
            pltpu.sync_copy(rows_v, out_hbm.at[pl.ds(off, chunk)])

    return k


_SIN_C = (0.99999970687168793, -0.16666577176372649, 0.0083325578491661046,
          -0.00019812568136934328, 2.7040424852228795e-06, -2.0533874739436797e-08)
_TWO_PI = 6.283185307179586
_INV_TWO_PI = 0.15915494309189535


def _fast_sin(x):
    r = x - jnp.round(x * _INV_TWO_PI) * _TWO_PI
    r2 = r * r
    p = _SIN_C[5]
    for cc in (_SIN_C[4], _SIN_C[3], _SIN_C[2], _SIN_C[1], _SIN_C[0]):
        p = p * r2 + cc
    return r * p


def _main_body(ef_ref, tm_ref, ga_ref, nf_ref,
               wall_ref, b_ref, wexp_ref, bexp_ref,
               p_ref, gf_ref, gt_ref,
               wat_ref, weon_ref, weo_ref, beo_ref, wnon_ref, wno_ref, bno_ref, out_ref):
    ef = ef_ref[...]
    tt = tm_ref[...][:, 0:T]
    mk = tm_ref[...][:, T:2 * T]
    be = ef.shape[0]

    tv_lin = jnp.dot(tt, wexp_ref[...], preferred_element_type=jnp.float32) + bexp_ref[...]
    lane = lax.broadcasted_iota(jnp.int32, (1, T * DT), 1)
    tv = jnp.where((lane % DT) < (DT - 1), _fast_sin(tv_lin), tv_lin)

    wall = wall_ref[...]
    bias = b_ref[...]
    h = jnp.zeros((be, 2 * H), jnp.float32)
    c = jnp.zeros((be, 2 * H), jnp.float32)
    h_last = jnp.zeros((be, 2 * H), jnp.float32)
    for t in range(T):
        xh = jnp.concatenate(
            [ef[:, EDGE_IN * t:EDGE_IN * (t + 1)], tv[:, DT * t:DT * (t + 1)], h], axis=1)
        g = jnp.dot(xh, wall, preferred_element_type=jnp.float32) + bias
        th = jnp.tanh(g)
        t_i = th[:, 0:128]
        t_f = th[:, 128:256]
        t_g = th[:, 256:384]
        t_o = th[:, 384:512]
        c = 0.5 * (c * (t_f + 1.0) + t_g * (t_i + 1.0))
        h = (t_o + 1.0) * jnp.tanh(c)
        h_last = jnp.where(mk[:, t:t + 1] > 0, h, h_last)

    e_out = h_last[:, 0:H]
    a_h = h_last[:, H:2 * H]
    a = jnp.dot(a_h, wat_ref[...], preferred_element_type=jnp.float32)
    a = jnp.where(a >= 0, a, 0.01 * a)
    m = jnp.maximum(
        jnp.dot(ga_ref[...], weon_ref[...], preferred_element_type=jnp.float32)
        + jnp.dot(e_out, weo_ref[...], preferred_element_type=jnp.float32)
        + beo_ref[...], 0.0)

    p_oh = p_ref[...]
    g_oh = gf_ref[...]
    gt_oh = gt_ref[...]

    z = jnp.dot(g_oh, a * p_oh, preferred_element_type=jnp.float32)
    z = z - jnp.max(z, axis=1, keepdims=True)
    cmp = (z[:, None, :] >= z[:, :, None]).astype(jnp.float32)
    r = jnp.sum(cmp, axis=2)
    s = jnp.sum(cmp * z[:, None, :], axis=2)
    valid = (1.0 + r * z) > s
    tauc = jnp.where(valid, (s - 1.0) / r, -1e30)
    tau = jnp.max(tauc, axis=1, keepdims=True)
    alpha = jnp.maximum(z - tau, 0.0)

    aedge = jnp.sum(jnp.dot(gt_oh, alpha, preferred_element_type=jnp.float32) * p_oh,
                    axis=1, keepdims=True)
    h_neigh = jnp.dot(g_oh, aedge * m, preferred_element_type=jnp.float32)
    out_ref[...] = jnp.maximum(
        jnp.dot(nf_ref[...], wnon_ref[...], preferred_element_type=jnp.float32)
        + jnp.dot(h_neigh, wno_ref[...], preferred_element_type=jnp.float32)
        + bno_ref[...], 0.0)


def _main_call(ef2, tm, ga, nf, wall, bias, wexp, bexp, p_oh, g_oh, gt_oh,
               wat, weon, weo, beo, wnon, wno, bno, bn, boff):
    be = DEG * bn

    def full(shape):
        return pl.BlockSpec(shape, lambda i: tuple(0 for _ in shape))

    nblk = ga.shape[0] // be
    b0 = boff
    return pl.pallas_call(
        _main_body,
        grid=(nblk,),
        in_specs=[
            pl.BlockSpec((be, T * EDGE_IN), lambda i: (i + b0, 0)),
            pl.BlockSpec((be, 2 * T), lambda i: (i + b0, 0)),
            pl.BlockSpec((be, NODE_IN), lambda i: (i, 0)),
            pl.BlockSpec((bn, NODE_IN), lambda i: (i + b0, 0)),
            full((2 * EDGE_IN + 2 * H, 8 * H)),
            full((1, 8 * H)),
            full((T, T * DT)),
            full((1, T * DT)),
            full((be, DEG)),
            full((bn, be)),
            full((be, bn)),
            full((H, 1)),
            full((NODE_IN, H)),
            full((H, H)),
            full((1, H)),
            full((NODE_IN, H)),
            full((H, H)),
            full((1, H)),
        ],
        out_specs=pl.BlockSpec((bn, H), lambda i: (i, 0)),
        out_shape=jax.ShapeDtypeStruct((nblk * bn, H), jnp.float32),
    )(ef2, tm, ga, nf, wall, bias, wexp, bexp, p_oh, g_oh, gt_oh,
      wat, weon, weo, beo, wnon, wno, bno)


def _interleave_gates(we, wa):
    blocks = []
    for k in range(4):
        blocks.append(we.T[:, H * k:H * (k + 1)])
        blocks.append(wa.T[:, H * k:H * (k + 1)])
    return jnp.concatenate(blocks, axis=1)


def kernel(node_features, edge_src, edge_features, edge_len, seq_times,
           t_w0, t_b0, t_w, t_b,
           e_wih, e_whh, e_bih, e_bhh,
           a_wih, a_whh, a_bih, a_bhh,
           w_attn, w_eout, b_eout, w_node, b_node):
    e = edge_features.shape[0]
    n = node_features.shape[0]

    ef2 = edge_features.reshape(e, T * EDGE_IN)
    mask = (jnp.arange(T, dtype=jnp.int32)[None, :] < edge_len[:, None]).astype(jnp.float32)
    tm = jnp.concatenate([seq_times, mask], axis=1)

    bn = 80
    be = DEG * bn
    p_oh = (jnp.arange(be, dtype=jnp.int32)[:, None] % DEG
            == jnp.arange(DEG, dtype=jnp.int32)[None, :]).astype(jnp.float32)
    g_oh = (jnp.arange(be, dtype=jnp.int32)[None, :] // DEG
            == jnp.arange(bn, dtype=jnp.int32)[:, None]).astype(jnp.float32)
    gt_oh = g_oh.T

    wrow = jnp.concatenate([t_w[0], t_w0[0]])
    wexp = jnp.kron(jnp.eye(T, dtype=jnp.float32), wrow[None, :])
    bexp = jnp.tile(jnp.concatenate([t_b, t_b0]), T)[None, :]

    wih_cat = _interleave_gates(e_wih, a_wih)
    z64 = jnp.zeros((H, H), jnp.float32)
    top, bot = [], []
    for k in range(4):
        top += [e_whh.T[:, H * k:H * (k + 1)], z64]
        bot += [z64, a_whh.T[:, H * k:H * (k + 1)]]
    whh_bd = jnp.concatenate(
        [jnp.concatenate(top, axis=1), jnp.concatenate(bot, axis=1)], axis=0)
    be_, ba_ = e_bih + e_bhh, a_bih + a_bhh
    bias = jnp.concatenate(
        [jnp.concatenate([be_[H * k:H * (k + 1)], ba_[H * k:H * (k + 1)]]) for k in range(4)]
    )[None, :]

    colscale = jnp.where((jnp.arange(8 * H) >= 4 * H) & (jnp.arange(8 * H) < 6 * H),
                         1.0, 0.5).astype(jnp.float32)[None, :]
    wih_cat = wih_cat * colscale
    whh_bd = whh_bd * colscale * 0.5
    bias = bias * colscale
    wall = jnp.concatenate([wih_cat, whh_bd], axis=0)

    weon = w_eout[:, :NODE_IN].T
    weo = w_eout[:, NODE_IN:].T * 0.5
    wnon = w_node[:, :NODE_IN].T
    wno = w_node[:, NODE_IN:].T
    watt = w_attn.T * 0.5

    args = (wall, bias, wexp, bexp, p_oh, g_oh, gt_oh,
            watt, weon, weo, b_eout[None, :], wnon, wno, b_node[None, :])
    be = DEG * bn
    gran = 32 * 1000
    e1 = ((e // be) // 2) * be
    if e1 >= gran:
        e1 = (e1 // gran) * gran
    g1 = _sc_gather(node_features, edge_src, e1, 0)
    g2 = _sc_gather(node_features, edge_src, e - e1, e1)
    o1 = _main_call(ef2, tm, g1, node_features, *args, bn=bn, boff=0)
    o2 = _main_call(ef2, tm, g2, node_features, *args, bn=bn, boff=e1 // (DEG * bn))
    return jnp.concatenate([o1, o2], axis=0)


def _sc_gather(table, idx, b, goff):
    return _make_sc_gather(table.shape[0], table.shape[1], b, goff)(table, idx)


# speedup vs baseline: 5.3632x; 1.0356x over previous
import functools

import jax
import jax.numpy as jnp
from jax import lax
from jax.experimental import pallas as pl
from jax.experimental.pallas import tpu as pltpu
from jax.experimental.pallas import tpu_sc as plsc

H = 64
T = 8
DEG = 16
EDGE_IN = 16
DT = 16
NODE_IN = 128


def _make_sc_gather(v, d, b, goff):
    info = plsc.get_sparse_core_info()
    nw = info.num_cores * info.num_subcores
    b_per_w = b // nw
    chunk = 1000
    n_chunks = b_per_w // chunk
    mesh = plsc.VectorSubcoreMesh(core_axis_name="c", subcore_axis_name="s")

    @functools.partial(
        pl.kernel,
        mesh=mesh,
        out_type=jax.ShapeDtypeStruct((b, d), jnp.float32),
        scratch_types=[
            pltpu.VMEM((chunk,), jnp.int32),
            pltpu.VMEM((chunk, d), jnp.float32),
            pltpu.SemaphoreType.DMA,
        ],
    )
    def k(table_hbm, idx_hbm, out_hbm, idx_v, rows_v, sem):
        wid = lax.axis_index("s") * info.num_cores + lax.axis_index("c")
        base = wid * b_per_w
        for j in range(n_chunks):
            off = base + j * chunk
            pltpu.sync_copy(idx_hbm.at[pl.ds(goff + off, chunk)], idx_v)
            pltpu.async_copy(table_hbm.at[idx_v], rows_v, sem).wait()
            pltpu.sync_copy(rows_v, out_hbm.at[pl.ds(off, chunk)])

    return k


_SIN_C = (0.99999970687168793, -0.16666577176372649, 0.0083325578491661046,
          -0.00019812568136934328, 2.7040424852228795e-06, -2.0533874739436797e-08)
_TWO_PI = 6.283185307179586
_INV_TWO_PI = 0.15915494309189535


def _fast_sin(x):
    r = x - jnp.round(x * _INV_TWO_PI) * _TWO_PI
    r2 = r * r
    p = _SIN_C[5]
    for cc in (_SIN_C[4], _SIN_C[3], _SIN_C[2], _SIN_C[1], _SIN_C[0]):
        p = p * r2 + cc
    return r * p


def _main_body(ef_ref, tm_ref, ga_ref, nf_ref,
               wall_ref, b_ref, wexp_ref, bexp_ref,
               p_ref, gf_ref, gt_ref,
               wat_ref, weon_ref, weo_ref, beo_ref, wnon_ref, wno_ref, bno_ref, out_ref):
    ef = ef_ref[...]
    tt = tm_ref[...][:, 0:T]
    mk = tm_ref[...][:, T:2 * T]
    be = ef.shape[0]

    tv_lin = jnp.dot(tt, wexp_ref[...], preferred_element_type=jnp.float32) + bexp_ref[...]
    lane = lax.broadcasted_iota(jnp.int32, (1, T * DT), 1)
    tv = jnp.where((lane % DT) < (DT - 1), _fast_sin(tv_lin), tv_lin)
    tv = tv.astype(jnp.bfloat16)

    wall = wall_ref[...]
    bias = b_ref[...]
    h = jnp.zeros((be, 2 * H), jnp.bfloat16)
    c = jnp.zeros((be, 2 * H), jnp.bfloat16)
    h_last = jnp.zeros((be, 2 * H), jnp.bfloat16)
    for t in range(T):
        xh = jnp.concatenate(
            [ef[:, EDGE_IN * t:EDGE_IN * (t + 1)], tv[:, DT * t:DT * (t + 1)], h], axis=1)
        g = jnp.dot(xh, wall, preferred_element_type=jnp.float32) + bias
        th = jnp.tanh(g.astype(jnp.bfloat16))
        t_i = th[:, 0:128]
        t_f = th[:, 128:256]
        t_g = th[:, 256:384]
        t_o = th[:, 384:512]
        c = 0.5 * (c * (t_f + 1.0) + t_g * (t_i + 1.0))
        h = (t_o + 1.0) * jnp.tanh(c)
        h_last = jnp.where(mk[:, t:t + 1] > 0, h, h_last)

    e_out = h_last[:, 0:H]
    a_h = h_last[:, H:2 * H]
    a = jnp.dot(a_h, wat_ref[...], preferred_element_type=jnp.float32)
    a = jnp.where(a >= 0, a, 0.01 * a)
    m = jnp.maximum(
        jnp.dot(ga_ref[...], weon_ref[...], preferred_element_type=jnp.float32)
        + jnp.dot(e_out, weo_ref[...], preferred_element_type=jnp.float32)
        + beo_ref[...], 0.0)

    p_oh = p_ref[...]
    g_oh = gf_ref[...]
    gt_oh = gt_ref[...]

    z = jnp.dot(g_oh, a * p_oh, preferred_element_type=jnp.float32)
    z = z - jnp.max(z, axis=1, keepdims=True)
    cmp = (z[:, None, :] >= z[:, :, None]).astype(jnp.float32)
    r = jnp.sum(cmp, axis=2)
    s = jnp.sum(cmp * z[:, None, :], axis=2)
    valid = (1.0 + r * z) > s
    tauc = jnp.where(valid, (s - 1.0) / r, -1e30)
    tau = jnp.max(tauc, axis=1, keepdims=True)
    alpha = jnp.maximum(z - tau, 0.0)

    aedge = jnp.sum(jnp.dot(gt_oh, alpha, preferred_element_type=jnp.float32) * p_oh,
                    axis=1, keepdims=True)
    h_neigh = jnp.dot(g_oh, aedge * m, preferred_element_type=jnp.float32)
    out_ref[...] = jnp.maximum(
        jnp.dot(nf_ref[...], wnon_ref[...], preferred_element_type=jnp.float32)
        + jnp.dot(h_neigh, wno_ref[...], preferred_element_type=jnp.float32)
        + bno_ref[...], 0.0)


def _main_call(ef2, tm, ga, nf, wall, bias, wexp, bexp, p_oh, g_oh, gt_oh,
               wat, weon, weo, beo, wnon, wno, bno, bn, boff):
    be = DEG * bn

    def full(shape):
        return pl.BlockSpec(shape, lambda i: tuple(0 for _ in shape))

    nblk = ga.shape[0] // be
    b0 = boff
    return pl.pallas_call(
        _main_body,
        grid=(nblk,),
        in_specs=[
            pl.BlockSpec((be, T * EDGE_IN), lambda i: (i + b0, 0)),
            pl.BlockSpec((be, 2 * T), lambda i: (i + b0, 0)),
            pl.BlockSpec((be, NODE_IN), lambda i: (i, 0)),
            pl.BlockSpec((bn, NODE_IN), lambda i: (i + b0, 0)),
            full((2 * EDGE_IN + 2 * H, 8 * H)),
            full((1, 8 * H)),
            full((T, T * DT)),
            full((1, T * DT)),
            full((be, DEG)),
            full((bn, be)),
            full((be, bn)),
            full((H, 1)),
            full((NODE_IN, H)),
            full((H, H)),
            full((1, H)),
            full((NODE_IN, H)),
            full((H, H)),
            full((1, H)),
        ],
        out_specs=pl.BlockSpec((bn, H), lambda i: (i, 0)),
        out_shape=jax.ShapeDtypeStruct((nblk * bn, H), jnp.float32),
    )(ef2, tm, ga, nf, wall, bias, wexp, bexp, p_oh, g_oh, gt_oh,
      wat, weon, weo, beo, wnon, wno, bno)


def _interleave_gates(we, wa):
    blocks = []
    for k in range(4):
        blocks.append(we.T[:, H * k:H * (k + 1)])
        blocks.append(wa.T[:, H * k:H * (k + 1)])
    return jnp.concatenate(blocks, axis=1)


def kernel(node_features, edge_src, edge_features, edge_len, seq_times,
           t_w0, t_b0, t_w, t_b,
           e_wih, e_whh, e_bih, e_bhh,
           a_wih, a_whh, a_bih, a_bhh,
           w_attn, w_eout, b_eout, w_node, b_node):
    e = edge_features.shape[0]
    n = node_features.shape[0]

    ef2 = edge_features.reshape(e, T * EDGE_IN).astype(jnp.bfloat16)
    mask = (jnp.arange(T, dtype=jnp.int32)[None, :] < edge_len[:, None]).astype(jnp.float32)
    tm = jnp.concatenate([seq_times, mask], axis=1)

    bn = 80
    be = DEG * bn
    p_oh = (jnp.arange(be, dtype=jnp.int32)[:, None] % DEG
            == jnp.arange(DEG, dtype=jnp.int32)[None, :]).astype(jnp.float32)
    g_oh = (jnp.arange(be, dtype=jnp.int32)[None, :] // DEG
            == jnp.arange(bn, dtype=jnp.int32)[:, None]).astype(jnp.float32)
    gt_oh = g_oh.T

    wrow = jnp.concatenate([t_w[0], t_w0[0]])
    wexp = jnp.kron(jnp.eye(T, dtype=jnp.float32), wrow[None, :])
    bexp = jnp.tile(jnp.concatenate([t_b, t_b0]), T)[None, :]

    wih_cat = _interleave_gates(e_wih, a_wih)
    z64 = jnp.zeros((H, H), jnp.float32)
    top, bot = [], []
    for k in range(4):
        top += [e_whh.T[:, H * k:H * (k + 1)], z64]
        bot += [z64, a_whh.T[:, H * k:H * (k + 1)]]
    whh_bd = jnp.concatenate(
        [jnp.concatenate(top, axis=1), jnp.concatenate(bot, axis=1)], axis=0)
    be_, ba_ = e_bih + e_bhh, a_bih + a_bhh
    bias = jnp.concatenate(
        [jnp.concatenate([be_[H * k:H * (k + 1)], ba_[H * k:H * (k + 1)]]) for k in range(4)]
    )[None, :]

    colscale = jnp.where((jnp.arange(8 * H) >= 4 * H) & (jnp.arange(8 * H) < 6 * H),
                         1.0, 0.5).astype(jnp.float32)[None, :]
    wih_cat = wih_cat * colscale
    whh_bd = whh_bd * colscale * 0.5
    bias = (bias * colscale).astype(jnp.bfloat16)
    wall = jnp.concatenate([wih_cat, whh_bd], axis=0).astype(jnp.bfloat16)

    weon = w_eout[:, :NODE_IN].T
    weo = (w_eout[:, NODE_IN:].T * 0.5).astype(jnp.bfloat16)
    wnon = w_node[:, :NODE_IN].T
    wno = w_node[:, NODE_IN:].T
    watt = (w_attn.T * 0.5).astype(jnp.bfloat16)

    gathered = _sc_gather(node_features, edge_src, e, 0)

    return _main_call(ef2, tm, gathered, node_features,
                      wall, bias, wexp, bexp, p_oh, g_oh, gt_oh,
                      watt, weon, weo, b_eout[None, :], wnon, wno, b_node[None, :],
                      bn=bn, boff=0)


def _sc_gather(table, idx, b, goff):
    return _make_sc_gather(table.shape[0], table.shape[1], b, goff)(table, idx)


# speedup vs baseline: 5.9421x; 1.1080x over previous
import functools

import jax
import jax.numpy as jnp
from jax import lax
from jax.experimental import pallas as pl
from jax.experimental.pallas import tpu as pltpu
from jax.experimental.pallas import tpu_sc as plsc

H = 64
T = 8
DEG = 16
EDGE_IN = 16
DT = 16
NODE_IN = 128


def _make_sc_gather(v, d, b, goff, dtype):
    info = plsc.get_sparse_core_info()
    nw = info.num_cores * info.num_subcores
    b_per_w = b // nw
    chunk = 200
    n_chunks = b_per_w // chunk
    mesh = plsc.VectorSubcoreMesh(core_axis_name="c", subcore_axis_name="s")

    @functools.partial(
        pl.kernel,
        mesh=mesh,
        out_type=jax.ShapeDtypeStruct((b, d), dtype),
        scratch_types=[
            pltpu.VMEM((chunk,), jnp.int32),
            pltpu.VMEM((chunk,), jnp.int32),
            pltpu.VMEM((chunk, d), dtype),
            pltpu.VMEM((chunk, d), dtype),
            pltpu.SemaphoreType.DMA,
            pltpu.SemaphoreType.DMA,
        ],
    )
    def k(table_hbm, idx_hbm, out_hbm, idx0, idx1, rows0, rows1, sem0, sem1):
        wid = lax.axis_index("s") * info.num_cores + lax.axis_index("c")
        base = wid * b_per_w
        idxs, rows, sems = (idx0, idx1), (rows0, rows1), (sem0, sem1)
        pltpu.sync_copy(idx_hbm.at[pl.ds(goff + base, chunk)], idxs[0])
        pend = pltpu.async_copy(table_hbm.at[idxs[0]], rows[0], sems[0])
        for j in range(n_chunks):
            nb = (j + 1) % 2
            cur = pend
            if j + 1 < n_chunks:
                off_n = base + (j + 1) * chunk
                pltpu.sync_copy(idx_hbm.at[pl.ds(goff + off_n, chunk)], idxs[nb])
                pend = pltpu.async_copy(table_hbm.at[idxs[nb]], rows[nb], sems[nb])
            cur.wait()
            pltpu.sync_copy(rows[j % 2], out_hbm.at[pl.ds(base + j * chunk, chunk)])

    return k


_SIN_C = (0.99999970687168793, -0.16666577176372649, 0.0083325578491661046,
          -0.00019812568136934328, 2.7040424852228795e-06, -2.0533874739436797e-08)
_TWO_PI = 6.283185307179586
_INV_TWO_PI = 0.15915494309189535


def _fast_sin(x):
    r = x - jnp.round(x * _INV_TWO_PI) * _TWO_PI
    r2 = r * r
    p = _SIN_C[5]
    for cc in (_SIN_C[4], _SIN_C[3], _SIN_C[2], _SIN_C[1], _SIN_C[0]):
        p = p * r2 + cc
    return r * p


def _main_body(ef_ref, tm_ref, ga_ref, nf_ref,
               wall_ref, b_ref, wexp_ref, bexp_ref,
               p_ref, gf_ref, gt_ref,
               wat_ref, weon_ref, weo_ref, beo_ref, wnon_ref, wno_ref, bno_ref, out_ref):
    ef = ef_ref[...]
    tt = tm_ref[...][:, 0:T]
    mk = tm_ref[...][:, T:2 * T]
    be = ef.shape[0]

    tv_lin = jnp.dot(tt, wexp_ref[...], preferred_element_type=jnp.float32) + bexp_ref[...]
    lane = lax.broadcasted_iota(jnp.int32, (1, T * DT), 1)
    tv = jnp.where((lane % DT) < (DT - 1), _fast_sin(tv_lin), tv_lin)
    tv = tv.astype(jnp.bfloat16)

    wall = wall_ref[...]
    bias = b_ref[...]
    h = jnp.zeros((be, 2 * H), jnp.bfloat16)
    c = jnp.zeros((be, 2 * H), jnp.bfloat16)
    h_last = jnp.zeros((be, 2 * H), jnp.bfloat16)
    for t in range(T):
        xh = jnp.concatenate(
            [ef[:, EDGE_IN * t:EDGE_IN * (t + 1)], tv[:, DT * t:DT * (t + 1)], h], axis=1)
        g = jnp.dot(xh, wall, preferred_element_type=jnp.float32)
        th = jnp.tanh(g.astype(jnp.bfloat16) + bias)
        t_i = th[:, 0:128]
        t_f = th[:, 128:256]
        t_g = th[:, 256:384]
        t_o = th[:, 384:512]
        c = 0.5 * (c * (t_f + 1.0) + t_g * (t_i + 1.0))
        h = (t_o + 1.0) * jnp.tanh(c)
        h_last = jnp.where(mk[:, t:t + 1] > 0, h, h_last)

    e_out = h_last[:, 0:H]
    a_h = h_last[:, H:2 * H]
    a = jnp.dot(a_h, wat_ref[...], preferred_element_type=jnp.float32)
    a = jnp.where(a >= 0, a, 0.01 * a)
    m = jnp.maximum(
        jnp.dot(ga_ref[...], weon_ref[...], preferred_element_type=jnp.float32)
        + jnp.dot(e_out, weo_ref[...], preferred_element_type=jnp.float32)
        + beo_ref[...], 0.0)

    p_oh = p_ref[...]
    g_oh = gf_ref[...]
    gt_oh = gt_ref[...]

    z = jnp.dot(g_oh, a * p_oh, preferred_element_type=jnp.float32)
    z = z - jnp.max(z, axis=1, keepdims=True)
    cmp = (z[:, None, :] >= z[:, :, None]).astype(jnp.float32)
    r = jnp.sum(cmp, axis=2)
    s = jnp.sum(cmp * z[:, None, :], axis=2)
    valid = (1.0 + r * z) > s
    tauc = jnp.where(valid, (s - 1.0) / r, -1e30)
    tau = jnp.max(tauc, axis=1, keepdims=True)
    alpha = jnp.maximum(z - tau, 0.0)

    aedge = jnp.sum(jnp.dot(gt_oh, alpha, preferred_element_type=jnp.float32) * p_oh,
                    axis=1, keepdims=True)
    h_neigh = jnp.dot(g_oh, aedge * m, preferred_element_type=jnp.float32)
    out_ref[...] = jnp.maximum(
        jnp.dot(nf_ref[...], wnon_ref[...], preferred_element_type=jnp.float32)
        + jnp.dot(h_neigh, wno_ref[...], preferred_element_type=jnp.float32)
        + bno_ref[...], 0.0)


def _main_call(ef2, tm, ga, nf, wall, bias, wexp, bexp, p_oh, g_oh, gt_oh,
               wat, weon, weo, beo, wnon, wno, bno, bn, boff):
    be = DEG * bn

    def full(shape):
        return pl.BlockSpec(shape, lambda i: tuple(0 for _ in shape))

    nblk = ga.shape[0] // be
    b0 = boff
    return pl.pallas_call(
        _main_body,
        grid=(nblk,),
        in_specs=[
            pl.BlockSpec((be, T * EDGE_IN), lambda i: (i + b0, 0)),
            pl.BlockSpec((be, 2 * T), lambda i: (i + b0, 0)),
            pl.BlockSpec((be, NODE_IN), lambda i: (i, 0)),
            pl.BlockSpec((bn, NODE_IN), lambda i: (i + b0, 0)),
            full((2 * EDGE_IN + 2 * H, 8 * H)),
            full((1, 8 * H)),
            full((T, T * DT)),
            full((1, T * DT)),
            full((be, DEG)),
            full((bn, be)),
            full((be, bn)),
            full((H, 1)),
            full((NODE_IN, H)),
            full((H, H)),
            full((1, H)),
            full((NODE_IN, H)),
            full((H, H)),
            full((1, H)),
        ],
        out_specs=pl.BlockSpec((bn, H), lambda i: (i, 0)),
        out_shape=jax.ShapeDtypeStruct((nblk * bn, H), jnp.float32),
    )(ef2, tm, ga, nf, wall, bias, wexp, bexp, p_oh, g_oh, gt_oh,
      wat, weon, weo, beo, wnon, wno, bno)


def _interleave_gates(we, wa):
    blocks = []
    for k in range(4):
        blocks.append(we.T[:, H * k:H * (k + 1)])
        blocks.append(wa.T[:, H * k:H * (k + 1)])
    return jnp.concatenate(blocks, axis=1)


def kernel(node_features, edge_src, edge_features, edge_len, seq_times,
           t_w0, t_b0, t_w, t_b,
           e_wih, e_whh, e_bih, e_bhh,
           a_wih, a_whh, a_bih, a_bhh,
           w_attn, w_eout, b_eout, w_node, b_node):
    e = edge_features.shape[0]
    n = node_features.shape[0]

    ef2 = edge_features.reshape(e, T * EDGE_IN).astype(jnp.bfloat16)
    mask = (jnp.arange(T, dtype=jnp.int32)[None, :] < edge_len[:, None]).astype(jnp.float32)
    tm = jnp.concatenate([seq_times, mask], axis=1)

    bn = 200
    be = DEG * bn
    p_oh = (jnp.arange(be, dtype=jnp.int32)[:, None] % DEG
            == jnp.arange(DEG, dtype=jnp.int32)[None, :]).astype(jnp.float32)
    g_oh = (jnp.arange(be, dtype=jnp.int32)[None, :] // DEG
            == jnp.arange(bn, dtype=jnp.int32)[:, None]).astype(jnp.float32)
    gt_oh = g_oh.T

    wrow = jnp.concatenate([t_w[0], t_w0[0]])
    wexp = jnp.kron(jnp.eye(T, dtype=jnp.float32), wrow[None, :])
    bexp = jnp.tile(jnp.concatenate([t_b, t_b0]), T)[None, :]

    wih_cat = _interleave_gates(e_wih, a_wih)
    z64 = jnp.zeros((H, H), jnp.float32)
    top, bot = [], []
    for k in range(4):
        top += [e_whh.T[:, H * k:H * (k + 1)], z64]
        bot += [z64, a_whh.T[:, H * k:H * (k + 1)]]
    whh_bd = jnp.concatenate(
        [jnp.concatenate(top, axis=1), jnp.concatenate(bot, axis=1)], axis=0)
    be_, ba_ = e_bih + e_bhh, a_bih + a_bhh
    bias = jnp.concatenate(
        [jnp.concatenate([be_[H * k:H * (k + 1)], ba_[H * k:H * (k + 1)]]) for k in range(4)]
    )[None, :]

    colscale = jnp.where((jnp.arange(8 * H) >= 4 * H) & (jnp.arange(8 * H) < 6 * H),
                         1.0, 0.5).astype(jnp.float32)[None, :]
    wih_cat = wih_cat * colscale
    whh_bd = whh_bd * colscale * 0.5
    bias = (bias * colscale).astype(jnp.bfloat16)
    wall = jnp.concatenate([wih_cat, whh_bd], axis=0).astype(jnp.bfloat16)

    weon = w_eout[:, :NODE_IN].T
    weo = (w_eout[:, NODE_IN:].T * 0.5).astype(jnp.bfloat16)
    wnon = w_node[:, :NODE_IN].T
    wno = w_node[:, NODE_IN:].T
    watt = (w_attn.T * 0.5).astype(jnp.bfloat16)

    gathered = _sc_gather(node_features, edge_src, e, 0)

    return _main_call(ef2, tm, gathered, node_features,
                      wall, bias, wexp, bexp, p_oh, g_oh, gt_oh,
                      watt, weon, weo, b_eout[None, :], wnon, wno, b_node[None, :],
                      bn=bn, boff=0)


def _sc_gather(table, idx, b, goff):
    return _make_sc_gather(table.shape[0], table.shape[1], b, goff, table.dtype)(table, idx)


# speedup vs baseline: 5.9950x; 1.0089x over previous
import functools

import jax
import jax.numpy as jnp
from jax import lax
from jax.experimental import pallas as pl
from jax.experimental.pallas import tpu as pltpu
from jax.experimental.pallas import tpu_sc as plsc

H = 64
T = 8
DEG = 16
EDGE_IN = 16
DT = 16
NODE_IN = 128


def _make_sc_gather(v, d, b, goff, dtype):
    info = plsc.get_sparse_core_info()
    nw = info.num_cores * info.num_subcores
    b_per_w = b // nw
    chunk = 1000
    n_chunks = b_per_w // chunk
    mesh = plsc.VectorSubcoreMesh(core_axis_name="c", subcore_axis_name="s")

    @functools.partial(
        pl.kernel,
        mesh=mesh,
        out_type=jax.ShapeDtypeStruct((b, d), dtype),
        scratch_types=[
            pltpu.VMEM((chunk,), jnp.int32),
            pltpu.VMEM((chunk, d), dtype),
            pltpu.SemaphoreType.DMA,
        ],
    )
    def k(table_hbm, idx_hbm, out_hbm, idx_v, rows_v, sem):
        wid = lax.axis_index("s") * info.num_cores + lax.axis_index("c")
        base = wid * b_per_w
        for j in range(n_chunks):
            off = base + j * chunk
            pltpu.sync_copy(idx_hbm.at[pl.ds(goff + off, chunk)], idx_v)
            pltpu.async_copy(table_hbm.at[idx_v], rows_v, sem).wait()
            pltpu.sync_copy(rows_v, out_hbm.at[pl.ds(off, chunk)])

    return k


_SIN_C = (0.99999970687168793, -0.16666577176372649, 0.0083325578491661046,
          -0.00019812568136934328, 2.7040424852228795e-06, -2.0533874739436797e-08)
_TWO_PI = 6.283185307179586
_INV_TWO_PI = 0.15915494309189535


def _fast_sin(x):
    r = x - jnp.round(x * _INV_TWO_PI) * _TWO_PI
    r2 = r * r
    p = _SIN_C[5]
    for cc in (_SIN_C[4], _SIN_C[3], _SIN_C[2], _SIN_C[1], _SIN_C[0]):
        p = p * r2 + cc
    return r * p


def _main_body(ef_ref, tm_ref, ga_ref, nf_ref,
               wall_ref, b_ref, wexp_ref, bexp_ref,
               p_ref, gf_ref, gt_ref,
               wat_ref, weon_ref, weo_ref, beo_ref, wnon_ref, wno_ref, bno_ref, out_ref):
    ef = ef_ref[...]
    tt = tm_ref[...][:, 0:T]
    mk = tm_ref[...][:, T:2 * T]
    be = ef.shape[0]

    tv_lin = jnp.dot(tt, wexp_ref[...], preferred_element_type=jnp.float32) + bexp_ref[...]
    lane = lax.broadcasted_iota(jnp.int32, (1, T * DT), 1)
    tv = jnp.where((lane % DT) < (DT - 1), _fast_sin(tv_lin), tv_lin)
    tv = tv.astype(jnp.bfloat16)

    wall = wall_ref[...]
    bias = b_ref[...]
    h = jnp.zeros((be, 2 * H), jnp.bfloat16)
    c = jnp.zeros((be, 2 * H), jnp.bfloat16)
    h_last = jnp.zeros((be, 2 * H), jnp.bfloat16)
    for t in range(T):
        xh = jnp.concatenate(
            [ef[:, EDGE_IN * t:EDGE_IN * (t + 1)], tv[:, DT * t:DT * (t + 1)], h], axis=1)
        g = jnp.dot(xh, wall, preferred_element_type=jnp.float32)
        th = jnp.tanh(g.astype(jnp.bfloat16) + bias)
        t_i = th[:, 0:128]
        t_f = th[:, 128:256]
        t_g = th[:, 256:384]
        t_o = th[:, 384:512]
        c = 0.5 * (c * (t_f + 1.0) + t_g * (t_i + 1.0))
        h = (t_o + 1.0) * jnp.tanh(c)
        h_last = jnp.where(mk[:, t:t + 1] > 0, h, h_last)

    e_out = h_last[:, 0:H]
    a_h = h_last[:, H:2 * H]
    a = jnp.dot(a_h, wat_ref[...], preferred_element_type=jnp.float32)
    a = jnp.where(a >= 0, a, 0.01 * a)
    m = jnp.maximum(
        jnp.dot(ga_ref[...], weon_ref[...], preferred_element_type=jnp.float32)
        + jnp.dot(e_out, weo_ref[...], preferred_element_type=jnp.float32)
        + beo_ref[...], 0.0)

    p_oh = p_ref[...]
    g_oh = gf_ref[...]
    gt_oh = gt_ref[...]

    z = jnp.dot(g_oh, a * p_oh, preferred_element_type=jnp.float32)
    z = z - jnp.max(z, axis=1, keepdims=True)
    cmp = (z[:, None, :] >= z[:, :, None]).astype(jnp.float32)
    r = jnp.sum(cmp, axis=2)
    s = jnp.sum(cmp * z[:, None, :], axis=2)
    valid = (1.0 + r * z) > s
    tauc = jnp.where(valid, (s - 1.0) / r, -1e30)
    tau = jnp.max(tauc, axis=1, keepdims=True)
    alpha = jnp.maximum(z - tau, 0.0)

    aedge = jnp.sum(jnp.dot(gt_oh, alpha, preferred_element_type=jnp.float32) * p_oh,
                    axis=1, keepdims=True)
    h_neigh = jnp.dot(g_oh, aedge * m, preferred_element_type=jnp.float32)
    out_ref[...] = jnp.maximum(
        jnp.dot(nf_ref[...], wnon_ref[...], preferred_element_type=jnp.float32)
        + jnp.dot(h_neigh, wno_ref[...], preferred_element_type=jnp.float32)
        + bno_ref[...], 0.0)


def _main_call(ef2, tm, ga, nf, wall, bias, wexp, bexp, p_oh, g_oh, gt_oh,
               wat, weon, weo, beo, wnon, wno, bno, bn, boff):
    be = DEG * bn

    def full(shape):
        return pl.BlockSpec(shape, lambda i: tuple(0 for _ in shape))

    nblk = ga.shape[0] // be
    b0 = boff
    return pl.pallas_call(
        _main_body,
        grid=(nblk,),
        in_specs=[
            pl.BlockSpec((be, T * EDGE_IN), lambda i: (i + b0, 0)),
            pl.BlockSpec((be, 2 * T), lambda i: (i + b0, 0)),
            pl.BlockSpec((be, NODE_IN), lambda i: (i, 0)),
            pl.BlockSpec((bn, NODE_IN), lambda i: (i + b0, 0)),
            full((2 * EDGE_IN + 2 * H, 8 * H)),
            full((1, 8 * H)),
            full((T, T * DT)),
            full((1, T * DT)),
            full((be, DEG)),
            full((bn, be)),
            full((be, bn)),
            full((H, 1)),
            full((NODE_IN, H)),
            full((H, H)),
            full((1, H)),
            full((NODE_IN, H)),
            full((H, H)),
            full((1, H)),
        ],
        out_specs=pl.BlockSpec((bn, H), lambda i: (i, 0)),
        out_shape=jax.ShapeDtypeStruct((nblk * bn, H), jnp.float32),
    )(ef2, tm, ga, nf, wall, bias, wexp, bexp, p_oh, g_oh, gt_oh,
      wat, weon, weo, beo, wnon, wno, bno)


def _interleave_gates(we, wa):
    blocks = []
    for k in range(4):
        blocks.append(we.T[:, H * k:H * (k + 1)])
        blocks.append(wa.T[:, H * k:H * (k + 1)])
    return jnp.concatenate(blocks, axis=1)


def kernel(node_features, edge_src, edge_features, edge_len, seq_times,
           t_w0, t_b0, t_w, t_b,
           e_wih, e_whh, e_bih, e_bhh,
           a_wih, a_whh, a_bih, a_bhh,
           w_attn, w_eout, b_eout, w_node, b_node):
    e = edge_features.shape[0]
    n = node_features.shape[0]

    ef2 = edge_features.reshape(e, T * EDGE_IN).astype(jnp.bfloat16)
    mask = (jnp.arange(T, dtype=jnp.int32)[None, :] < edge_len[:, None]).astype(jnp.float32)
    tm = jnp.concatenate([seq_times, mask], axis=1)

    bn = 200
    be = DEG * bn
    p_oh = (jnp.arange(be, dtype=jnp.int32)[:, None] % DEG
            == jnp.arange(DEG, dtype=jnp.int32)[None, :]).astype(jnp.float32)
    g_oh = (jnp.arange(be, dtype=jnp.int32)[None, :] // DEG
            == jnp.arange(bn, dtype=jnp.int32)[:, None]).astype(jnp.float32)
    gt_oh = g_oh.T

    wrow = jnp.concatenate([t_w[0], t_w0[0]])
    wexp = jnp.kron(jnp.eye(T, dtype=jnp.float32), wrow[None, :])
    bexp = jnp.tile(jnp.concatenate([t_b, t_b0]), T)[None, :]

    wih_cat = _interleave_gates(e_wih, a_wih)
    z64 = jnp.zeros((H, H), jnp.float32)
    top, bot = [], []
    for k in range(4):
        top += [e_whh.T[:, H * k:H * (k + 1)], z64]
        bot += [z64, a_whh.T[:, H * k:H * (k + 1)]]
    whh_bd = jnp.concatenate(
        [jnp.concatenate(top, axis=1), jnp.concatenate(bot, axis=1)], axis=0)
    be_, ba_ = e_bih + e_bhh, a_bih + a_bhh
    bias = jnp.concatenate(
        [jnp.concatenate([be_[H * k:H * (k + 1)], ba_[H * k:H * (k + 1)]]) for k in range(4)]
    )[None, :]

    colscale = jnp.where((jnp.arange(8 * H) >= 4 * H) & (jnp.arange(8 * H) < 6 * H),
                         1.0, 0.5).astype(jnp.float32)[None, :]
    wih_cat = wih_cat * colscale
    whh_bd = whh_bd * colscale * 0.5
    bias = (bias * colscale).astype(jnp.bfloat16)
    wall = jnp.concatenate([wih_cat, whh_bd], axis=0).astype(jnp.bfloat16)

    weon = w_eout[:, :NODE_IN].T
    weo = (w_eout[:, NODE_IN:].T * 0.5).astype(jnp.bfloat16)
    wnon = w_node[:, :NODE_IN].T
    wno = w_node[:, NODE_IN:].T
    watt = (w_attn.T * 0.5).astype(jnp.bfloat16)

    gathered = _sc_gather(node_features, edge_src, e, 0)

    return _main_call(ef2, tm, gathered, node_features,
                      wall, bias, wexp, bexp, p_oh, g_oh, gt_oh,
                      watt, weon, weo, b_eout[None, :], wnon, wno, b_node[None, :],
                      bn=bn, boff=0)


def _sc_gather(table, idx, b, goff):
    return _make_sc_gather(table.shape[0], table.shape[1], b, goff, table.dtype)(table, idx)
